# baseline (speedup 1.0000x reference)
import functools

import jax
import jax.numpy as jnp
from jax import lax
from jax.experimental import pallas as pl
from jax.experimental.pallas import tpu as pltpu
from jax.experimental.pallas import tpu_sc as plsc

N = 10000
E = 320000
F_IN = 128
H = 128
C = 40

NP_ = 10240
NC, NS = 2, 16
NW = NC * NS
CH = 128
NB = 2
NCHR = 40
NPASS = 2
NCH = NCHR * NPASS
EPT = CH * NCH
EP = NW * EPT
RPT = NP_ // NS
RB = 512
GRID = NP_ // RB


def _agg_kernel(h, src2, dst2, zrows, s_out, srcv, dstv, gbuf, acc, gsem,
                ssem):
    c = lax.axis_index("c")
    s = lax.axis_index("s")
    wid = s * NC + c

    r0 = s * RPT
    for k in range(RPT // 128):
        pltpu.sync_copy(zrows, acc.at[pl.ds(r0 + k * 128, 128)])
    plsc.subcore_barrier()

    def start_g(j, b):
        pltpu.make_async_copy(h.at[srcv.at[j]], gbuf.at[b], gsem.at[b]).start()

    def wait_g(j, b):
        pltpu.make_async_copy(h.at[srcv.at[j]], gbuf.at[b], gsem.at[b]).wait()

    def start_s(j, b):
        pltpu.async_copy(gbuf.at[b], acc.at[dstv.at[j]], ssem.at[b], add=True)

    def wait_s(j, b):
        pltpu.make_async_copy(gbuf.at[b], acc.at[dstv.at[j]], ssem.at[b]).wait()

    for t in range(NPASS):
        pltpu.sync_copy(src2.at[pl.ds(wid * NCH + t * NCHR, NCHR)], srcv)
        pltpu.sync_copy(dst2.at[pl.ds(wid * NCH + t * NCHR, NCHR)], dstv)
        for b in range(NB):
            start_g(b, b)

        @pl.loop(0, NCHR - NB, step=NB)
        def _(j0):
            for b in range(NB):
                wait_g(j0 + b, b)
                start_s(j0 + b, b)
            for b in range(NB):
                wait_s(j0 + b, b)
                start_g(j0 + NB + b, b)

        for b in range(NB):
            wait_g(NCHR - NB + b, b)
            start_s(NCHR - NB + b, b)
        for b in range(NB):
            wait_s(NCHR - NB + b, b)

    plsc.subcore_barrier()

    o0 = c * NP_ + s * RPT
    pltpu.sync_copy(acc.at[pl.ds(s * RPT, RPT)], s_out.at[pl.ds(o0, RPT)])


def _deg_kernel(dst2, orows, zrows, d_out, dstv, onesv, dacc, ssem):
    c = lax.axis_index("c")
    s = lax.axis_index("s")
    wid = s * NC + c

    pltpu.sync_copy(orows, onesv)
    r0 = s * RPT
    for k in range(RPT // 128):
        pltpu.sync_copy(zrows, dacc.at[pl.ds(r0 + k * 128, 128)])
    plsc.subcore_barrier()

    def start_s(j, b):
        pltpu.async_copy(onesv, dacc.at[dstv.at[j]], ssem.at[b], add=True)

    def wait_s(j, b):
        pltpu.make_async_copy(onesv, dacc.at[dstv.at[j]], ssem.at[b]).wait()

    for t in range(NPASS):
        pltpu.sync_copy(dst2.at[pl.ds(wid * NCH + t * NCHR, NCHR)], dstv)
        for b in range(NB):
            start_s(b, b)

        @pl.loop(0, NCHR - NB, step=NB)
        def _(j0):
            for b in range(NB):
                wait_s(j0 + b, b)
                start_s(j0 + NB + b, b)

        for b in range(NB):
            wait_s(NCHR - NB + b, b)

    plsc.subcore_barrier()
    o0 = c * NP_ + s * RPT
    pltpu.sync_copy(dacc.at[pl.ds(s * RPT, RPT)], d_out.at[pl.ds(o0, RPT)])


def _sc_deg(dst2, orows, zrows):
    mesh = plsc.VectorSubcoreMesh(core_axis_name="c", subcore_axis_name="s", num_cores=NC, num_subcores=NS)
    return pl.kernel(
        _deg_kernel,
        out_type=jax.ShapeDtypeStruct((NC * NP_, H), jnp.float32),
        mesh=mesh,
        scratch_types=[
            pltpu.VMEM((NCHR, CH), jnp.int32),
            pltpu.VMEM((CH, H), jnp.float32),
            pltpu.VMEM_SHARED((NP_, H), jnp.float32),
            pltpu.SemaphoreType.DMA((NB,)),
        ],
    )(dst2, orows, zrows)


def _sc_agg(h, src2, dst2, zrows):
    mesh = plsc.VectorSubcoreMesh(core_axis_name="c", subcore_axis_name="s", num_cores=NC, num_subcores=NS)
    return pl.kernel(
        _agg_kernel,
        out_type=jax.ShapeDtypeStruct((NC * NP_, H), jnp.float32),
        mesh=mesh,
        scratch_types=[
            pltpu.VMEM((NCHR, CH), jnp.int32),
            pltpu.VMEM((NCHR, CH), jnp.int32),
            pltpu.VMEM((NB, CH, H), jnp.float32),
            pltpu.VMEM_SHARED((NP_, H), jnp.float32),
            pltpu.SemaphoreType.DMA((NB,)),
            pltpu.SemaphoreType.DMA((NB,)),
        ],
    )(h, src2, dst2, zrows)


def _row_spec(i_ofs=0):
    def im(i):
        return (i + i_ofs, 0)
    return pl.BlockSpec((RB, H), im)


def _full_spec(shape):
    nd = len(shape)
    return pl.BlockSpec(shape, lambda i: (0,) * nd)


def _encode_body(x_ref, wp_ref, wq_ref, p_ref, q_ref):
    xv = x_ref[...]
    p_ref[...] = jnp.dot(xv, wp_ref[...], preferred_element_type=jnp.float32)
    q_ref[...] = jnp.dot(xv, wq_ref[...], preferred_element_type=jnp.float32)


def _encode(x, wp, wq):
    return pl.pallas_call(
        _encode_body,
        grid=(GRID,),
        in_specs=[pl.BlockSpec((RB, F_IN), lambda i: (i, 0)),
                  _full_spec((F_IN, H)), _full_spec((F_IN, H))],
        out_specs=[_row_spec(), _row_spec()],
        out_shape=[jax.ShapeDtypeStruct((NP_, H), jnp.float32)] * 2,
    )(x, wp, wq)


def _up_first_body(a0, a1, d0, d1, p_ref, w_ref, pn_ref, rdeg_ref):
    deg = jnp.maximum(d0[:, 0:1] + d1[:, 0:1], 1.0)
    rdeg = lax.broadcast_in_dim(1.0 / deg, (RB, H), (0, 1))
    rdeg_ref[...] = rdeg
    t = (a0[...] + a1[...]) * rdeg
    pn_ref[...] = p_ref[...] + jnp.dot(t, w_ref[...],
                                       preferred_element_type=jnp.float32)


def _up_first(S, D, p, w):
    return pl.pallas_call(
        _up_first_body,
        grid=(GRID,),
        in_specs=[_row_spec(), _row_spec(GRID),
                  _row_spec(), _row_spec(GRID),
                  _row_spec(), _full_spec((H, H))],
        out_specs=[_row_spec(), _row_spec()],
        out_shape=[jax.ShapeDtypeStruct((NP_, H), jnp.float32)] * 2,
    )(S, S, D, D, p, w)


def _up_body(a0, a1, rdeg, p_ref, w_ref, pn_ref):
    t = (a0[...] + a1[...]) * rdeg[...]
    pn_ref[...] = p_ref[...] + jnp.dot(t, w_ref[...],
                                       preferred_element_type=jnp.float32)


def _up(S, rdeg, p, w):
    return pl.pallas_call(
        _up_body,
        grid=(GRID,),
        in_specs=[_row_spec(), _row_spec(GRID), _row_spec(), _row_spec(),
                  _full_spec((H, H))],
        out_specs=_row_spec(),
        out_shape=jax.ShapeDtypeStruct((NP_, H), jnp.float32),
    )(S, S, rdeg, p, w)


def _down_act_body(b0, b1, rdeg, q_ref, p_ref, w_ref, au_ref, ad_ref,
                   pn_ref, qn_ref):
    t = (b0[...] + b1[...]) * rdeg[...]
    q1 = q_ref[...] + jnp.dot(t, w_ref[...], preferred_element_type=jnp.float32)
    pn = p_ref[...] + au_ref[...] * jnp.tanh(q1)
    pn_ref[...] = pn
    qn_ref[...] = q1 + ad_ref[...] * jnp.tanh(pn)


def _down_act(S, rdeg, q, p, w, au, ad):
    return pl.pallas_call(
        _down_act_body,
        grid=(GRID,),
        in_specs=[_row_spec(), _row_spec(GRID), _row_spec(), _row_spec(),
                  _row_spec(), _full_spec((H, H)),
                  _full_spec((H,)), _full_spec((H,))],
        out_specs=[_row_spec(), _row_spec()],
        out_shape=[jax.ShapeDtypeStruct((NP_, H), jnp.float32)] * 2,
    )(S, S, rdeg, q, p, w, au, ad)


def _final_body(b0, b1, rdeg, q_ref, w_ref, wo_ref, bo_ref, o_ref):
    t = (b0[...] + b1[...]) * rdeg[...]
    q1 = q_ref[...] + jnp.dot(t, w_ref[...], preferred_element_type=jnp.float32)
    o_ref[...] = (jnp.dot(q1, wo_ref[...], preferred_element_type=jnp.float32)
                  + bo_ref[...])


def _final(S, rdeg, q, w, wo, bo):
    return pl.pallas_call(
        _final_body,
        grid=(GRID,),
        in_specs=[_row_spec(), _row_spec(GRID), _row_spec(), _row_spec(),
                  _full_spec((H, H)), _full_spec((H, C)), _full_spec((C,))],
        out_specs=pl.BlockSpec((RB, C), lambda i: (i, 0)),
        out_shape=jax.ShapeDtypeStruct((NP_, C), jnp.float32),
    )(S, S, rdeg, q, w, wo, bo)


def kernel(x, edge_index, Wp, Wq, up_W0, down_W0, a_up0, a_down0,
           up_W1, down_W1, a_up1, a_down1, up_W2, down_W2, W_out, b_out):
    xp = jnp.zeros((NP_, F_IN), jnp.float32).at[:N].set(x)
    pad = N + jnp.arange(EP - E, dtype=jnp.int32) % (NP_ - N)
    src2 = jnp.concatenate([edge_index[0], pad]).reshape(NW * NCH, CH)
    dst2 = jnp.concatenate([edge_index[1], pad]).reshape(NW * NCH, CH)
    zrows = jnp.zeros((128, H), jnp.float32)
    orows = jnp.ones((CH, H), jnp.float32)

    p, q = _encode(xp, Wp, Wq)
    D = _sc_deg(dst2, orows, zrows)
    S = _sc_agg(q, src2, dst2, zrows)
    p, rdeg = _up_first(S, D, p, up_W0)
    S = _sc_agg(p, src2, dst2, zrows)
    p, q = _down_act(S, rdeg, q, p, down_W0, a_up0, a_down0)
    S = _sc_agg(q, src2, dst2, zrows)
    p = _up(S, rdeg, p, up_W1)
    S = _sc_agg(p, src2, dst2, zrows)
    p, q = _down_act(S, rdeg, q, p, down_W1, a_up1, a_down1)
    S = _sc_agg(q, src2, dst2, zrows)
    p = _up(S, rdeg, p, up_W2)
    S = _sc_agg(p, src2, dst2, zrows)
    out = _final(S, rdeg, q, down_W2, W_out, b_out)
    return out[:N]


# speedup vs baseline: 1.0933x; 1.0933x over previous
import functools

import jax
import jax.numpy as jnp
from jax import lax
from jax.experimental import pallas as pl
from jax.experimental.pallas import tpu as pltpu
from jax.experimental.pallas import tpu_sc as plsc

N = 10000
E = 320000
F_IN = 128
H = 128
C = 40

NP_ = 10240
NC, NS = 2, 16
NW = NC * NS
CH = 32
NB = 8
NCHR = 40
NPASS = 8
NCH = NCHR * NPASS
EPT = CH * NCH
EP = NW * EPT
RPT = NP_ // NS
RB = 512
GRID = NP_ // RB


def _agg_kernel(h, src2, dst2, zrows, s_out, srcv, dstv, gbuf, acc, gsem,
                ssem):
    c = lax.axis_index("c")
    s = lax.axis_index("s")
    wid = s * NC + c

    r0 = s * RPT
    for k in range(RPT // 128):
        pltpu.sync_copy(zrows, acc.at[pl.ds(r0 + k * 128, 128)])
    plsc.subcore_barrier()

    def start_g(j, b):
        pltpu.make_async_copy(h.at[srcv.at[j]], gbuf.at[b], gsem.at[b]).start()

    def wait_g(j, b):
        pltpu.make_async_copy(h.at[srcv.at[j]], gbuf.at[b], gsem.at[b]).wait()

    def start_s(j, b):
        pltpu.async_copy(gbuf.at[b], acc.at[dstv.at[j]], ssem.at[b], add=True)

    def wait_s(j, b):
        pltpu.make_async_copy(gbuf.at[b], acc.at[dstv.at[j]], ssem.at[b]).wait()

    for t in range(NPASS):
        pltpu.sync_copy(src2.at[pl.ds(wid * NCH + t * NCHR, NCHR)], srcv)
        pltpu.sync_copy(dst2.at[pl.ds(wid * NCH + t * NCHR, NCHR)], dstv)
        for b in range(NB):
            start_g(b, b)

        @pl.loop(0, NCHR - NB, step=NB)
        def _(j0):
            for b in range(NB):
                wait_g(j0 + b, b)
                start_s(j0 + b, b)
            for b in range(NB):
                wait_s(j0 + b, b)
                start_g(j0 + NB + b, b)

        for b in range(NB):
            wait_g(NCHR - NB + b, b)
            start_s(NCHR - NB + b, b)
        for b in range(NB):
            wait_s(NCHR - NB + b, b)

    plsc.subcore_barrier()

    o0 = c * NP_ + s * RPT
    pltpu.sync_copy(acc.at[pl.ds(s * RPT, RPT)], s_out.at[pl.ds(o0, RPT)])


def _deg_kernel(dst2, orows, zrows, d_out, dstv, onesv, dacc, ssem):
    c = lax.axis_index("c")
    s = lax.axis_index("s")
    wid = s * NC + c

    pltpu.sync_copy(orows, onesv)
    r0 = s * RPT
    for k in range(RPT // 128):
        pltpu.sync_copy(zrows, dacc.at[pl.ds(r0 + k * 128, 128)])
    plsc.subcore_barrier()

    def start_s(j, b):
        pltpu.async_copy(onesv, dacc.at[dstv.at[j]], ssem.at[b], add=True)

    def wait_s(j, b):
        pltpu.make_async_copy(onesv, dacc.at[dstv.at[j]], ssem.at[b]).wait()

    for t in range(NPASS):
        pltpu.sync_copy(dst2.at[pl.ds(wid * NCH + t * NCHR, NCHR)], dstv)
        for b in range(NB):
            start_s(b, b)

        @pl.loop(0, NCHR - NB, step=NB)
        def _(j0):
            for b in range(NB):
                wait_s(j0 + b, b)
                start_s(j0 + NB + b, b)

        for b in range(NB):
            wait_s(NCHR - NB + b, b)

    plsc.subcore_barrier()
    o0 = c * NP_ + s * RPT
    pltpu.sync_copy(dacc.at[pl.ds(s * RPT, RPT)], d_out.at[pl.ds(o0, RPT)])


def _sc_deg(dst2, orows, zrows):
    mesh = plsc.VectorSubcoreMesh(core_axis_name="c", subcore_axis_name="s", num_cores=NC, num_subcores=NS)
    return pl.kernel(
        _deg_kernel,
        out_type=jax.ShapeDtypeStruct((NC * NP_, H), jnp.float32),
        mesh=mesh,
        scratch_types=[
            pltpu.VMEM((NCHR, CH), jnp.int32),
            pltpu.VMEM((CH, H), jnp.float32),
            pltpu.VMEM_SHARED((NP_, H), jnp.float32),
            pltpu.SemaphoreType.DMA((NB,)),
        ],
    )(dst2, orows, zrows)


def _sc_agg(h, src2, dst2, zrows):
    mesh = plsc.VectorSubcoreMesh(core_axis_name="c", subcore_axis_name="s", num_cores=NC, num_subcores=NS)
    return pl.kernel(
        _agg_kernel,
        out_type=jax.ShapeDtypeStruct((NC * NP_, H), jnp.float32),
        mesh=mesh,
        scratch_types=[
            pltpu.VMEM((NCHR, CH), jnp.int32),
            pltpu.VMEM((NCHR, CH), jnp.int32),
            pltpu.VMEM((NB, CH, H), jnp.float32),
            pltpu.VMEM_SHARED((NP_, H), jnp.float32),
            pltpu.SemaphoreType.DMA((NB,)),
            pltpu.SemaphoreType.DMA((NB,)),
        ],
    )(h, src2, dst2, zrows)


def _row_spec(i_ofs=0):
    def im(i):
        return (i + i_ofs, 0)
    return pl.BlockSpec((RB, H), im)


def _full_spec(shape):
    nd = len(shape)
    return pl.BlockSpec(shape, lambda i: (0,) * nd)


def _encode_body(x_ref, wp_ref, wq_ref, p_ref, q_ref):
    xv = x_ref[...]
    p_ref[...] = jnp.dot(xv, wp_ref[...], preferred_element_type=jnp.float32)
    q_ref[...] = jnp.dot(xv, wq_ref[...], preferred_element_type=jnp.float32)


def _encode(x, wp, wq):
    return pl.pallas_call(
        _encode_body,
        grid=(GRID,),
        in_specs=[pl.BlockSpec((RB, F_IN), lambda i: (i, 0)),
                  _full_spec((F_IN, H)), _full_spec((F_IN, H))],
        out_specs=[_row_spec(), _row_spec()],
        out_shape=[jax.ShapeDtypeStruct((NP_, H), jnp.float32)] * 2,
    )(x, wp, wq)


def _up_first_body(a0, a1, d0, d1, p_ref, w_ref, pn_ref, rdeg_ref):
    deg = jnp.maximum(d0[:, 0:1] + d1[:, 0:1], 1.0)
    rdeg = lax.broadcast_in_dim(1.0 / deg, (RB, H), (0, 1))
    rdeg_ref[...] = rdeg
    t = (a0[...] + a1[...]) * rdeg
    pn_ref[...] = p_ref[...] + jnp.dot(t, w_ref[...],
                                       preferred_element_type=jnp.float32)


def _up_first(S, D, p, w):
    return pl.pallas_call(
        _up_first_body,
        grid=(GRID,),
        in_specs=[_row_spec(), _row_spec(GRID),
                  _row_spec(), _row_spec(GRID),
                  _row_spec(), _full_spec((H, H))],
        out_specs=[_row_spec(), _row_spec()],
        out_shape=[jax.ShapeDtypeStruct((NP_, H), jnp.float32)] * 2,
    )(S, S, D, D, p, w)


def _up_body(a0, a1, rdeg, p_ref, w_ref, pn_ref):
    t = (a0[...] + a1[...]) * rdeg[...]
    pn_ref[...] = p_ref[...] + jnp.dot(t, w_ref[...],
                                       preferred_element_type=jnp.float32)


def _up(S, rdeg, p, w):
    return pl.pallas_call(
        _up_body,
        grid=(GRID,),
        in_specs=[_row_spec(), _row_spec(GRID), _row_spec(), _row_spec(),
                  _full_spec((H, H))],
        out_specs=_row_spec(),
        out_shape=jax.ShapeDtypeStruct((NP_, H), jnp.float32),
    )(S, S, rdeg, p, w)


def _down_act_body(b0, b1, rdeg, q_ref, p_ref, w_ref, au_ref, ad_ref,
                   pn_ref, qn_ref):
    t = (b0[...] + b1[...]) * rdeg[...]
    q1 = q_ref[...] + jnp.dot(t, w_ref[...], preferred_element_type=jnp.float32)
    pn = p_ref[...] + au_ref[...] * jnp.tanh(q1)
    pn_ref[...] = pn
    qn_ref[...] = q1 + ad_ref[...] * jnp.tanh(pn)


def _down_act(S, rdeg, q, p, w, au, ad):
    return pl.pallas_call(
        _down_act_body,
        grid=(GRID,),
        in_specs=[_row_spec(), _row_spec(GRID), _row_spec(), _row_spec(),
                  _row_spec(), _full_spec((H, H)),
                  _full_spec((H,)), _full_spec((H,))],
        out_specs=[_row_spec(), _row_spec()],
        out_shape=[jax.ShapeDtypeStruct((NP_, H), jnp.float32)] * 2,
    )(S, S, rdeg, q, p, w, au, ad)


def _final_body(b0, b1, rdeg, q_ref, w_ref, wo_ref, bo_ref, o_ref):
    t = (b0[...] + b1[...]) * rdeg[...]
    q1 = q_ref[...] + jnp.dot(t, w_ref[...], preferred_element_type=jnp.float32)
    o_ref[...] = (jnp.dot(q1, wo_ref[...], preferred_element_type=jnp.float32)
                  + bo_ref[...])


def _final(S, rdeg, q, w, wo, bo):
    return pl.pallas_call(
        _final_body,
        grid=(GRID,),
        in_specs=[_row_spec(), _row_spec(GRID), _row_spec(), _row_spec(),
                  _full_spec((H, H)), _full_spec((H, C)), _full_spec((C,))],
        out_specs=pl.BlockSpec((RB, C), lambda i: (i, 0)),
        out_shape=jax.ShapeDtypeStruct((NP_, C), jnp.float32),
    )(S, S, rdeg, q, w, wo, bo)


def kernel(x, edge_index, Wp, Wq, up_W0, down_W0, a_up0, a_down0,
           up_W1, down_W1, a_up1, a_down1, up_W2, down_W2, W_out, b_out):
    xp = jnp.zeros((NP_, F_IN), jnp.float32).at[:N].set(x)
    pad = N + jnp.arange(EP - E, dtype=jnp.int32) % (NP_ - N)
    src2 = jnp.concatenate([edge_index[0], pad]).reshape(NW * NCH, CH)
    dst2 = jnp.concatenate([edge_index[1], pad]).reshape(NW * NCH, CH)
    zrows = jnp.zeros((128, H), jnp.float32)
    orows = jnp.ones((CH, H), jnp.float32)

    p, q = _encode(xp, Wp, Wq)
    D = _sc_deg(dst2, orows, zrows)
    S = _sc_agg(q, src2, dst2, zrows)
    p, rdeg = _up_first(S, D, p, up_W0)
    S = _sc_agg(p, src2, dst2, zrows)
    p, q = _down_act(S, rdeg, q, p, down_W0, a_up0, a_down0)
    S = _sc_agg(q, src2, dst2, zrows)
    p = _up(S, rdeg, p, up_W1)
    S = _sc_agg(p, src2, dst2, zrows)
    p, q = _down_act(S, rdeg, q, p, down_W1, a_up1, a_down1)
    S = _sc_agg(q, src2, dst2, zrows)
    p = _up(S, rdeg, p, up_W2)
    S = _sc_agg(p, src2, dst2, zrows)
    out = _final(S, rdeg, q, down_W2, W_out, b_out)
    return out[:N]


# speedup vs baseline: 1.2229x; 1.1185x over previous
import functools

import jax
import jax.numpy as jnp
from jax import lax
from jax.experimental import pallas as pl
from jax.experimental.pallas import tpu as pltpu
from jax.experimental.pallas import tpu_sc as plsc

N = 10000
E = 320000
F_IN = 128
H = 128
C = 40

NP_ = 10240
NC, NS = 2, 16
NW = NC * NS
CH = 64
NB = 4
NCHR = 40
NPASS = 4
NCH = NCHR * NPASS
EPT = CH * NCH
EP = NW * EPT
RPT = NP_ // NS
RB = 2048
GRID = NP_ // RB


def _agg_kernel(h, src2, dst2, zrows, s_out, srcv, dstv, gbuf, acc, gsem,
                ssem):
    c = lax.axis_index("c")
    s = lax.axis_index("s")
    wid = s * NC + c

    r0 = s * RPT
    for k in range(RPT // 128):
        pltpu.sync_copy(zrows, acc.at[pl.ds(r0 + k * 128, 128)])
    plsc.subcore_barrier()

    def start_g(j, b):
        pltpu.make_async_copy(h.at[srcv.at[j]], gbuf.at[b], gsem.at[b]).start()

    def wait_g(j, b):
        pltpu.make_async_copy(h.at[srcv.at[j]], gbuf.at[b], gsem.at[b]).wait()

    def start_s(j, b):
        pltpu.async_copy(gbuf.at[b], acc.at[dstv.at[j]], ssem.at[b], add=True)

    def wait_s(j, b):
        pltpu.make_async_copy(gbuf.at[b], acc.at[dstv.at[j]], ssem.at[b]).wait()

    for t in range(NPASS):
        pltpu.sync_copy(src2.at[pl.ds(wid * NCH + t * NCHR, NCHR)], srcv)
        pltpu.sync_copy(dst2.at[pl.ds(wid * NCH + t * NCHR, NCHR)], dstv)
        for b in range(NB):
            start_g(b, b)

        @pl.loop(0, NCHR - NB, step=NB)
        def _(j0):
            for b in range(NB):
                wait_g(j0 + b, b)
                start_s(j0 + b, b)
            for b in range(NB):
                wait_s(j0 + b, b)
                start_g(j0 + NB + b, b)

        for b in range(NB):
            wait_g(NCHR - NB + b, b)
            start_s(NCHR - NB + b, b)
        for b in range(NB):
            wait_s(NCHR - NB + b, b)

    plsc.subcore_barrier()

    o0 = c * NP_ + s * RPT
    pltpu.sync_copy(acc.at[pl.ds(s * RPT, RPT)], s_out.at[pl.ds(o0, RPT)])


def _deg_kernel(dst2, orows, zrows, d_out, dstv, onesv, dacc, ssem):
    c = lax.axis_index("c")
    s = lax.axis_index("s")
    wid = s * NC + c

    pltpu.sync_copy(orows, onesv)
    r0 = s * RPT
    for k in range(RPT // 128):
        pltpu.sync_copy(zrows, dacc.at[pl.ds(r0 + k * 128, 128)])
    plsc.subcore_barrier()

    def start_s(j, b):
        pltpu.async_copy(onesv, dacc.at[dstv.at[j]], ssem.at[b], add=True)

    def wait_s(j, b):
        pltpu.make_async_copy(onesv, dacc.at[dstv.at[j]], ssem.at[b]).wait()

    for t in range(NPASS):
        pltpu.sync_copy(dst2.at[pl.ds(wid * NCH + t * NCHR, NCHR)], dstv)
        for b in range(NB):
            start_s(b, b)

        @pl.loop(0, NCHR - NB, step=NB)
        def _(j0):
            for b in range(NB):
                wait_s(j0 + b, b)
                start_s(j0 + NB + b, b)

        for b in range(NB):
            wait_s(NCHR - NB + b, b)

    plsc.subcore_barrier()
    o0 = c * NP_ + s * RPT
    pltpu.sync_copy(dacc.at[pl.ds(s * RPT, RPT)], d_out.at[pl.ds(o0, RPT)])


def _sc_deg(dst2, orows, zrows):
    mesh = plsc.VectorSubcoreMesh(core_axis_name="c", subcore_axis_name="s", num_cores=NC, num_subcores=NS)
    return pl.kernel(
        _deg_kernel,
        out_type=jax.ShapeDtypeStruct((NC * NP_, H), jnp.float32),
        mesh=mesh,
        scratch_types=[
            pltpu.VMEM((NCHR, CH), jnp.int32),
            pltpu.VMEM((CH, H), jnp.float32),
            pltpu.VMEM_SHARED((NP_, H), jnp.float32),
            pltpu.SemaphoreType.DMA((NB,)),
        ],
    )(dst2, orows, zrows)


def _sc_agg(h, src2, dst2, zrows):
    mesh = plsc.VectorSubcoreMesh(core_axis_name="c", subcore_axis_name="s", num_cores=NC, num_subcores=NS)
    return pl.kernel(
        _agg_kernel,
        out_type=jax.ShapeDtypeStruct((NC * NP_, H), jnp.float32),
        mesh=mesh,
        scratch_types=[
            pltpu.VMEM((NCHR, CH), jnp.int32),
            pltpu.VMEM((NCHR, CH), jnp.int32),
            pltpu.VMEM((NB, CH, H), jnp.float32),
            pltpu.VMEM_SHARED((NP_, H), jnp.float32),
            pltpu.SemaphoreType.DMA((NB,)),
            pltpu.SemaphoreType.DMA((NB,)),
        ],
    )(h, src2, dst2, zrows)


def _row_spec(i_ofs=0):
    def im(i):
        return (i + i_ofs, 0)
    return pl.BlockSpec((RB, H), im)


def _full_spec(shape):
    nd = len(shape)
    return pl.BlockSpec(shape, lambda i: (0,) * nd)


def _encode_body(x_ref, wp_ref, wq_ref, p_ref, q_ref):
    xv = x_ref[...]
    p_ref[...] = jnp.dot(xv, wp_ref[...], preferred_element_type=jnp.float32)
    q_ref[...] = jnp.dot(xv, wq_ref[...], preferred_element_type=jnp.float32)


def _encode(x, wp, wq):
    return pl.pallas_call(
        _encode_body,
        grid=(GRID,),
        in_specs=[pl.BlockSpec((RB, F_IN), lambda i: (i, 0)),
                  _full_spec((F_IN, H)), _full_spec((F_IN, H))],
        out_specs=[_row_spec(), _row_spec()],
        out_shape=[jax.ShapeDtypeStruct((NP_, H), jnp.float32)] * 2,
    )(x, wp, wq)


def _up_first_body(a0, a1, d0, d1, p_ref, w_ref, pn_ref, rdeg_ref):
    deg = jnp.maximum(d0[:, 0:1] + d1[:, 0:1], 1.0)
    rdeg = lax.broadcast_in_dim(1.0 / deg, (RB, H), (0, 1))
    rdeg_ref[...] = rdeg
    t = (a0[...] + a1[...]) * rdeg
    pn_ref[...] = p_ref[...] + jnp.dot(t, w_ref[...],
                                       preferred_element_type=jnp.float32)


def _up_first(S, D, p, w):
    return pl.pallas_call(
        _up_first_body,
        grid=(GRID,),
        in_specs=[_row_spec(), _row_spec(GRID),
                  _row_spec(), _row_spec(GRID),
                  _row_spec(), _full_spec((H, H))],
        out_specs=[_row_spec(), _row_spec()],
        out_shape=[jax.ShapeDtypeStruct((NP_, H), jnp.float32)] * 2,
    )(S, S, D, D, p, w)


def _up_body(a0, a1, rdeg, p_ref, w_ref, pn_ref):
    t = (a0[...] + a1[...]) * rdeg[...]
    pn_ref[...] = p_ref[...] + jnp.dot(t, w_ref[...],
                                       preferred_element_type=jnp.float32)


def _up(S, rdeg, p, w):
    return pl.pallas_call(
        _up_body,
        grid=(GRID,),
        in_specs=[_row_spec(), _row_spec(GRID), _row_spec(), _row_spec(),
                  _full_spec((H, H))],
        out_specs=_row_spec(),
        out_shape=jax.ShapeDtypeStruct((NP_, H), jnp.float32),
    )(S, S, rdeg, p, w)


def _down_act_body(b0, b1, rdeg, q_ref, p_ref, w_ref, au_ref, ad_ref,
                   pn_ref, qn_ref):
    t = (b0[...] + b1[...]) * rdeg[...]
    q1 = q_ref[...] + jnp.dot(t, w_ref[...], preferred_element_type=jnp.float32)
    pn = p_ref[...] + au_ref[...] * jnp.tanh(q1)
    pn_ref[...] = pn
    qn_ref[...] = q1 + ad_ref[...] * jnp.tanh(pn)


def _down_act(S, rdeg, q, p, w, au, ad):
    return pl.pallas_call(
        _down_act_body,
        grid=(GRID,),
        in_specs=[_row_spec(), _row_spec(GRID), _row_spec(), _row_spec(),
                  _row_spec(), _full_spec((H, H)),
                  _full_spec((H,)), _full_spec((H,))],
        out_specs=[_row_spec(), _row_spec()],
        out_shape=[jax.ShapeDtypeStruct((NP_, H), jnp.float32)] * 2,
    )(S, S, rdeg, q, p, w, au, ad)


def _final_body(b0, b1, rdeg, q_ref, w_ref, wo_ref, bo_ref, o_ref):
    t = (b0[...] + b1[...]) * rdeg[...]
    q1 = q_ref[...] + jnp.dot(t, w_ref[...], preferred_element_type=jnp.float32)
    o_ref[...] = (jnp.dot(q1, wo_ref[...], preferred_element_type=jnp.float32)
                  + bo_ref[...])


def _final(S, rdeg, q, w, wo, bo):
    return pl.pallas_call(
        _final_body,
        grid=(GRID,),
        in_specs=[_row_spec(), _row_spec(GRID), _row_spec(), _row_spec(),
                  _full_spec((H, H)), _full_spec((H, C)), _full_spec((C,))],
        out_specs=pl.BlockSpec((RB, C), lambda i: (i, 0)),
        out_shape=jax.ShapeDtypeStruct((NP_, C), jnp.float32),
    )(S, S, rdeg, q, w, wo, bo)


def kernel(x, edge_index, Wp, Wq, up_W0, down_W0, a_up0, a_down0,
           up_W1, down_W1, a_up1, a_down1, up_W2, down_W2, W_out, b_out):
    xp = jnp.zeros((NP_, F_IN), jnp.float32).at[:N].set(x)
    pad = N + jnp.arange(EP - E, dtype=jnp.int32) % (NP_ - N)
    src2 = jnp.concatenate([edge_index[0], pad]).reshape(NW * NCH, CH)
    dst2 = jnp.concatenate([edge_index[1], pad]).reshape(NW * NCH, CH)
    zrows = jnp.zeros((128, H), jnp.float32)
    orows = jnp.ones((CH, H), jnp.float32)

    p, q = _encode(xp, Wp, Wq)
    D = _sc_deg(dst2, orows, zrows)
    S = _sc_agg(q, src2, dst2, zrows)
    p, rdeg = _up_first(S, D, p, up_W0)
    S = _sc_agg(p, src2, dst2, zrows)
    p, q = _down_act(S, rdeg, q, p, down_W0, a_up0, a_down0)
    S = _sc_agg(q, src2, dst2, zrows)
    p = _up(S, rdeg, p, up_W1)
    S = _sc_agg(p, src2, dst2, zrows)
    p, q = _down_act(S, rdeg, q, p, down_W1, a_up1, a_down1)
    S = _sc_agg(q, src2, dst2, zrows)
    p = _up(S, rdeg, p, up_W2)
    S = _sc_agg(p, src2, dst2, zrows)
    out = _final(S, rdeg, q, down_W2, W_out, b_out)
    return out[:N]
